# baseline (speedup 1.0000x reference)
import jax
import jax.numpy as jnp
from jax.experimental import pallas as pl


def _add_body(x_ref, p_ref, o_ref):
    o_ref[...] = x_ref[...] + p_ref[...]


def _blocked_add(xf, pos_table):
    T, D = pos_table.shape
    N = xf.shape[0]
    return pl.pallas_call(
        _add_body,
        grid=(N // T,),
        in_specs=[
            pl.BlockSpec((T, D), lambda i: (i, 0)),
            pl.BlockSpec((T, D), lambda i: (0, 0)),
        ],
        out_specs=pl.BlockSpec((T, D), lambda i: (i, 0)),
        out_shape=jax.ShapeDtypeStruct((N, D), xf.dtype),
    )(xf, pos_table)


def kernel(x, pos_table):
    T, D = pos_table.shape
    xf = x.reshape(-1, D)
    N = xf.shape[0]
    split = (N // T - 1) * T
    out0 = _blocked_add(xf[:split], pos_table)
    out1 = _blocked_add(xf[split:], pos_table)
    return jnp.concatenate([out0, out1], axis=0).reshape(-1, T, D)
